# Initial kernel scaffold; baseline (speedup 1.0000x reference)
import jax
import jax.numpy as jnp
from jax.experimental import pallas as pl


def kernel(feat, roi_boxxes_batch):
    raise NotImplementedError("write your pallas kernel here")


# speedup vs baseline: 3.1682x; 3.1682x over previous
import functools

import jax
import jax.numpy as jnp
from jax import lax
from jax.experimental import pallas as pl
from jax.experimental.pallas import tpu as pltpu
from jax.experimental.pallas import tpu_sc as plsc

P = 8
B, L, D, N = 4, 4096, 256, 1000
NPAD = 1024
RPW = NPAD // 8
WIN = 80
NCHUNK = D // 16

_INV = [1.0, 0.5, 1.0 / 3.0, 0.25, 0.2, 1.0 / 6.0, 1.0 / 7.0, 0.125]


def _inv_small_int(g):
  inv = jnp.float32(_INV[7])
  for k in range(6, -1, -1):
    inv = jnp.where(g == k + 1, jnp.float32(_INV[k]), inv)
  return inv


def _body(starts_hbm, ends_hbm, feat_hbm, out_hbm, win_v, out_v,
          box_sp, s_sm, e_sm):
  cid = lax.axis_index("c")
  sid = lax.axis_index("s")
  wid = cid * 16 + sid
  b = wid // 8
  base = (wid % 8) * RPW

  pltpu.sync_copy(starts_hbm.at[b, pl.ds(base, RPW)], box_sp.at[sid, 0])
  pltpu.sync_copy(ends_hbm.at[b, pl.ds(base, RPW)], box_sp.at[sid, 1])
  pltpu.sync_copy(box_sp.at[sid, 0], s_sm)
  pltpu.sync_copy(box_sp.at[sid, 1], e_sm)

  def roi_body(j, carry):
    s = s_sm[j]
    e = e_sm[j]
    roi_start = s - 0.5
    roi_h = e - s
    bin_h = roi_h * jnp.float32(1.0 / P)
    gi = bin_h.astype(jnp.int32)
    g0 = gi + (gi.astype(jnp.float32) < bin_h).astype(jnp.int32)
    g0 = jnp.maximum(g0, 1)
    inv_cnt = _inv_small_int(g0)
    step = bin_h * inv_cnt
    rs0 = jnp.maximum(roi_start, 0.0)
    w0 = rs0.astype(jnp.int32)
    w0 = w0 - (w0.astype(jnp.float32) > rs0).astype(jnp.int32)
    w0 = (w0 // 8) * 8
    w0 = jnp.minimum(w0, L - WIN)

    pltpu.sync_copy(feat_hbm.at[b, pl.ds(w0, WIN)], win_v)

    for ph in range(P):
      y0 = roi_start + jnp.float32(ph) * bin_h + jnp.float32(0.5) * step

      accs = [jnp.zeros((16,), jnp.float32) for _ in range(NCHUNK)]
      for iy in range(8):
        m = (iy < g0).astype(jnp.float32)
        y = y0 + jnp.float32(iy) * step
        yc = jnp.minimum(jnp.maximum(y, 0.0), jnp.float32(L - 1))
        ylow = yc.astype(jnp.int32)
        ylow = ylow - (ylow.astype(jnp.float32) > yc).astype(jnp.int32)
        ly = yc - ylow.astype(jnp.float32)
        hy = m - ly * m
        ly = ly * m
        rel = ylow - w0
        relh = jnp.minimum(rel + 1, WIN - 1)
        for c in range(NCHUNK):
          vl = win_v[rel, pl.ds(c * 16, 16)]
          vh = win_v[relh, pl.ds(c * 16, 16)]
          accs[c] = accs[c] + hy * vl + ly * vh
      for c in range(NCHUNK):
        out_v[ph, pl.ds(c * 16, 16)] = accs[c] * inv_cnt

    pltpu.sync_copy(out_v, out_hbm.at[b, base + j])
    return carry

  lax.fori_loop(0, RPW, roi_body, 0)


@jax.jit
def kernel(feat, roi_boxxes_batch):
  starts = roi_boxxes_batch[..., 0]
  ends = roi_boxxes_batch[..., 1]
  pad = NPAD - N
  starts = jnp.pad(starts, ((0, 0), (0, pad)))
  ends = jnp.pad(ends, ((0, 0), (0, pad)), constant_values=8.0)

  mesh = plsc.VectorSubcoreMesh(
      core_axis_name="c", subcore_axis_name="s", num_cores=2, num_subcores=16)
  run = pl.kernel(
      _body,
      out_type=jax.ShapeDtypeStruct((B, NPAD, P, D), jnp.float32),
      mesh=mesh,
      scratch_types=[
          pltpu.VMEM((WIN, D), jnp.float32),
          pltpu.VMEM((P, D), jnp.float32),
          pltpu.VMEM_SHARED((16, 2, RPW), jnp.float32),
          pltpu.SMEM((RPW,), jnp.float32),
          pltpu.SMEM((RPW,), jnp.float32),
      ],
  )
  out = run(starts, ends, feat)
  return out[:, :N]


# speedup vs baseline: 11.6725x; 3.6843x over previous
import jax
import jax.numpy as jnp
from jax import lax
from jax.experimental import pallas as pl
from jax.experimental.pallas import tpu as pltpu
from jax.experimental.pallas import tpu_sc as plsc

P = 8
B, L, D, N = 4, 4096, 256, 1000
NPAD = 1024
RPW = NPAD // 8
WIN = 80
NCHUNK = D // 16

_INV = [1.0, 0.5, 1.0 / 3.0, 0.25, 0.2, 1.0 / 6.0, 1.0 / 7.0, 0.125]


def _inv_small_int(g):
  inv = jnp.float32(_INV[7])
  for k in range(6, -1, -1):
    inv = jnp.where(g == k + 1, jnp.float32(_INV[k]), inv)
  return inv


def _floor_nonneg(x):
  f = x.astype(jnp.int32)
  return f - (f.astype(jnp.float32) > x).astype(jnp.int32)


def _window_start(s):
  rs0 = jnp.maximum(s - 0.5, 0.0)
  w0 = (_floor_nonneg(rs0) // 8) * 8
  return jnp.minimum(w0, L - WIN)


def _body(starts_hbm, ends_hbm, feat_hbm, out_hbm, win_v, out_v,
          box_sp, s_sm, e_sm, sem0, sem1):
  cid = lax.axis_index("c")
  sid = lax.axis_index("s")
  wid = cid * 16 + sid
  b = wid // 8
  base = (wid % 8) * RPW

  pltpu.sync_copy(starts_hbm.at[b, pl.ds(base, RPW)], box_sp.at[sid, 0])
  pltpu.sync_copy(ends_hbm.at[b, pl.ds(base, RPW)], box_sp.at[sid, 1])
  pltpu.sync_copy(box_sp.at[sid, 0], s_sm)
  pltpu.sync_copy(box_sp.at[sid, 1], e_sm)

  sems = (sem0, sem1)

  def issue(j, buf):
    w0 = _window_start(s_sm[j])
    pltpu.make_async_copy(
        feat_hbm.at[b, pl.ds(w0, WIN)], win_v.at[buf], sems[buf]).start()

  def wait(buf):
    pltpu.make_async_copy(
        feat_hbm.at[b, pl.ds(0, WIN)], win_v.at[buf], sems[buf]).wait()

  def compute(j, buf):
    s = s_sm[j]
    e = e_sm[j]
    roi_start = s - 0.5
    roi_h = e - s
    bin_h = roi_h * jnp.float32(1.0 / P)
    gi = bin_h.astype(jnp.int32)
    g0 = gi + (gi.astype(jnp.float32) < bin_h).astype(jnp.int32)
    g0 = jnp.maximum(g0, 1)
    inv_cnt = _inv_small_int(g0)
    step = bin_h * inv_cnt
    w0 = _window_start(s)

    for ph in range(P):
      y0 = roi_start + jnp.float32(ph) * bin_h + jnp.float32(0.5) * step

      def samp(iy, accs):
        y = y0 + iy.astype(jnp.float32) * step
        yc = jnp.minimum(jnp.maximum(y, 0.0), jnp.float32(L - 1))
        ylow = _floor_nonneg(yc)
        ly = yc - ylow.astype(jnp.float32)
        hy = 1.0 - ly
        rel = ylow - w0
        relh = jnp.minimum(rel + 1, WIN - 1)
        new = []
        for c in range(NCHUNK):
          vl = win_v[buf, rel, pl.ds(c * 16, 16)]
          vh = win_v[buf, relh, pl.ds(c * 16, 16)]
          new.append(accs[c] + hy * vl + ly * vh)
        return tuple(new)

      accs0 = tuple(jnp.zeros((16,), jnp.float32) for _ in range(NCHUNK))
      accs = lax.fori_loop(0, g0, samp, accs0)
      for c in range(NCHUNK):
        out_v[ph, pl.ds(c * 16, 16)] = accs[c] * inv_cnt

    pltpu.sync_copy(out_v, out_hbm.at[b, base + j])

  issue(jnp.int32(0), 0)

  def pair(t, carry):
    j0 = 2 * t
    issue(j0 + 1, 1)
    wait(0)
    compute(j0, 0)

    @pl.when(j0 + 2 < RPW)
    def _():
      issue(j0 + 2, 0)

    wait(1)
    compute(j0 + 1, 1)
    return carry

  lax.fori_loop(0, RPW // 2, pair, 0)


@jax.jit
def kernel(feat, roi_boxxes_batch):
  starts = roi_boxxes_batch[..., 0]
  ends = roi_boxxes_batch[..., 1]
  pad = NPAD - N
  starts = jnp.pad(starts, ((0, 0), (0, pad)))
  ends = jnp.pad(ends, ((0, 0), (0, pad)), constant_values=8.0)

  mesh = plsc.VectorSubcoreMesh(
      core_axis_name="c", subcore_axis_name="s", num_cores=2, num_subcores=16)
  run = pl.kernel(
      _body,
      out_type=jax.ShapeDtypeStruct((B, NPAD, P, D), jnp.float32),
      mesh=mesh,
      scratch_types=[
          pltpu.VMEM((2, WIN, D), jnp.float32),
          pltpu.VMEM((P, D), jnp.float32),
          pltpu.VMEM_SHARED((16, 2, RPW), jnp.float32),
          pltpu.SMEM((RPW,), jnp.float32),
          pltpu.SMEM((RPW,), jnp.float32),
          pltpu.SemaphoreType.DMA,
          pltpu.SemaphoreType.DMA,
      ],
  )
  out = run(starts, ends, feat)
  return out[:, :N]
